# Initial kernel scaffold; baseline (speedup 1.0000x reference)
import jax
import jax.numpy as jnp
from jax.experimental import pallas as pl


def kernel(pred_r, pred_t, target, model_points, idx):
    raise NotImplementedError("write your pallas kernel here")


# speedup vs baseline: 2.0946x; 2.0946x over previous
import functools

import jax
import jax.numpy as jnp
from jax.experimental import pallas as pl
from jax.experimental.pallas import tpu as pltpu

_B = 64
_N = 3000
_NPAD = 3072
_IB = 768
_NIB = _NPAD // _IB
_SYM = (12, 15, 18, 19, 20)
_PAD_COORD = 1.0e6


def _loss_kernel(mask_ref, rt_ref, mp_ref, tT_ref, trows_ref, out_ref):
    b = pl.program_id(0)
    i = pl.program_id(1)

    mp = mp_ref[0]
    mx, my, mz = mp[:, 0:1], mp[:, 1:2], mp[:, 2:3]

    tfx = mx * rt_ref[b, 0] + my * rt_ref[b, 3] + mz * rt_ref[b, 6] + rt_ref[b, 9]
    tfy = mx * rt_ref[b, 1] + my * rt_ref[b, 4] + mz * rt_ref[b, 7] + rt_ref[b, 10]
    tfz = mx * rt_ref[b, 2] + my * rt_ref[b, 5] + mz * rt_ref[b, 8] + rt_ref[b, 11]

    row_id = i * _IB + jax.lax.broadcasted_iota(jnp.int32, (_IB, 1), 0)
    row_ok = row_id < _N

    @pl.when(i == 0)
    def _init():
        out_ref[...] = jnp.zeros((1, 1, 1), jnp.float32)

    is_sym = mask_ref[b] != 0

    @pl.when(is_sym)
    def _sym_path():
        ones = jnp.ones_like(tfx)
        a = jnp.concatenate([-2.0 * tfx, -2.0 * tfy, -2.0 * tfz, ones], axis=1)
        tT = tT_ref[0]
        r2 = (tT[0:1, :] * tT[0:1, :] + tT[1:2, :] * tT[1:2, :]
              + tT[2:3, :] * tT[2:3, :])
        bm = jnp.concatenate([tT, r2], axis=0)
        d2m = jnp.dot(a, bm, preferred_element_type=jnp.float32,
                      precision=jax.lax.Precision.HIGHEST)
        q2 = tfx * tfx + tfy * tfy + tfz * tfz
        dmin = jnp.min(d2m, axis=1, keepdims=True) + q2
        dis = jnp.sqrt(jnp.maximum(dmin, 0.0))
        out_ref[...] += jnp.sum(jnp.where(row_ok, dis, 0.0)).reshape(1, 1, 1) / _N

    @pl.when(jnp.logical_not(is_sym))
    def _direct_path():
        tr = trows_ref[0]
        dx = tfx - tr[:, 0:1]
        dy = tfy - tr[:, 1:2]
        dz = tfz - tr[:, 2:3]
        dis = jnp.sqrt(dx * dx + dy * dy + dz * dz)
        out_ref[...] += jnp.sum(jnp.where(row_ok, dis, 0.0)).reshape(1, 1, 1) / _N


@jax.jit
def _run(mask, rt, mp_pad, tT, t_pad):
    grid_spec = pltpu.PrefetchScalarGridSpec(
        num_scalar_prefetch=2,
        grid=(_B, _NIB),
        in_specs=[
            pl.BlockSpec((1, _IB, 3), lambda b, i, m, r: (b, i, 0)),
            pl.BlockSpec((1, 3, _NPAD), lambda b, i, m, r: (b, 0, 0)),
            pl.BlockSpec((1, _IB, 3), lambda b, i, m, r: (b, i, 0)),
        ],
        out_specs=pl.BlockSpec((1, 1, 1), lambda b, i, m, r: (b, 0, 0)),
    )
    return pl.pallas_call(
        _loss_kernel,
        grid_spec=grid_spec,
        out_shape=jax.ShapeDtypeStruct((_B, 1, 1), jnp.float32),
        compiler_params=pltpu.CompilerParams(
            dimension_semantics=("arbitrary", "arbitrary"),
        ),
    )(mask, rt, mp_pad, tT, t_pad)


def kernel(pred_r, pred_t, target, model_points, idx):
    pred_r = pred_r / jnp.linalg.norm(pred_r, axis=1, keepdims=True)
    w, x, y, z = pred_r[:, 0], pred_r[:, 1], pred_r[:, 2], pred_r[:, 3]
    rt = jnp.stack([
        1.0 - 2.0 * (y * y + z * z), 2.0 * (x * y - w * z), 2.0 * (x * z + w * y),
        2.0 * (x * y + w * z), 1.0 - 2.0 * (x * x + z * z), 2.0 * (y * z - w * x),
        2.0 * (x * z - w * y), 2.0 * (y * z + w * x), 1.0 - 2.0 * (x * x + y * y),
        pred_t[:, 0], pred_t[:, 1], pred_t[:, 2],
    ], axis=1)
    rt = jnp.pad(rt, ((0, 0), (0, 4)))

    sym = jnp.asarray(_SYM, dtype=idx.dtype)
    mask = (idx[:, 0][:, None] == sym[None, :]).any(axis=1).astype(jnp.int32)

    mp_pad = jnp.pad(model_points, ((0, 0), (0, _NPAD - _N), (0, 0)))
    t_pad = jnp.pad(target, ((0, 0), (0, _NPAD - _N), (0, 0)),
                    constant_values=_PAD_COORD)
    tT = jnp.transpose(t_pad, (0, 2, 1))

    out = _run(mask, rt, mp_pad, tT, t_pad)
    return out[:, 0, 0]


# speedup vs baseline: 2.2418x; 1.0703x over previous
import jax
import jax.numpy as jnp
from jax.experimental import pallas as pl
from jax.experimental.pallas import tpu as pltpu

_B = 64
_N = 3000
_NPAD = 3072
_JT = 768
_NJT = _NPAD // _JT
_SYM = (12, 15, 18, 19, 20)
_PAD_COORD = 1.0e6


def _loss_kernel(mask_ref, rt_ref, mp_ref, tT_ref, trows_ref, out_ref):
    b = pl.program_id(0)

    mp = mp_ref[0]
    mx, my, mz = mp[:, 0:1], mp[:, 1:2], mp[:, 2:3]

    tfx = mx * rt_ref[b, 0] + my * rt_ref[b, 3] + mz * rt_ref[b, 6] + rt_ref[b, 9]
    tfy = mx * rt_ref[b, 1] + my * rt_ref[b, 4] + mz * rt_ref[b, 7] + rt_ref[b, 10]
    tfz = mx * rt_ref[b, 2] + my * rt_ref[b, 5] + mz * rt_ref[b, 8] + rt_ref[b, 11]

    row_id = jax.lax.broadcasted_iota(jnp.int32, (_NPAD, 1), 0)
    row_ok = row_id < _N

    is_sym = mask_ref[b] != 0

    @pl.when(is_sym)
    def _sym_path():
        ones = jnp.ones_like(tfx)
        a = jnp.concatenate([-2.0 * tfx, -2.0 * tfy, -2.0 * tfz, ones], axis=1)
        q2 = tfx * tfx + tfy * tfy + tfz * tfz
        dmin = None
        for j in range(_NJT):
            tT = tT_ref[0, :, j * _JT:(j + 1) * _JT]
            r2 = (tT[0:1, :] * tT[0:1, :] + tT[1:2, :] * tT[1:2, :]
                  + tT[2:3, :] * tT[2:3, :])
            bm = jnp.concatenate([tT, r2], axis=0)
            d2m = jnp.dot(a, bm, preferred_element_type=jnp.float32,
                          precision=jax.lax.Precision.HIGHEST)
            tmin = jnp.min(d2m, axis=1, keepdims=True)
            dmin = tmin if dmin is None else jnp.minimum(dmin, tmin)
        dis = jnp.sqrt(jnp.maximum(dmin + q2, 0.0))
        out_ref[...] = jnp.sum(jnp.where(row_ok, dis, 0.0)).reshape(1, 1, 1) / _N

    @pl.when(jnp.logical_not(is_sym))
    def _direct_path():
        tr = trows_ref[0]
        dx = tfx - tr[:, 0:1]
        dy = tfy - tr[:, 1:2]
        dz = tfz - tr[:, 2:3]
        dis = jnp.sqrt(dx * dx + dy * dy + dz * dz)
        out_ref[...] = jnp.sum(jnp.where(row_ok, dis, 0.0)).reshape(1, 1, 1) / _N


@jax.jit
def _run(mask, rt, mp_pad, tT, t_pad):
    grid_spec = pltpu.PrefetchScalarGridSpec(
        num_scalar_prefetch=2,
        grid=(_B,),
        in_specs=[
            pl.BlockSpec((1, _NPAD, 3), lambda b, m, r: (b, 0, 0)),
            pl.BlockSpec((1, 3, _NPAD), lambda b, m, r: (b, 0, 0)),
            pl.BlockSpec((1, _NPAD, 3), lambda b, m, r: (b, 0, 0)),
        ],
        out_specs=pl.BlockSpec((1, 1, 1), lambda b, m, r: (b, 0, 0)),
    )
    return pl.pallas_call(
        _loss_kernel,
        grid_spec=grid_spec,
        out_shape=jax.ShapeDtypeStruct((_B, 1, 1), jnp.float32),
        compiler_params=pltpu.CompilerParams(
            dimension_semantics=("arbitrary",),
        ),
    )(mask, rt, mp_pad, tT, t_pad)


def kernel(pred_r, pred_t, target, model_points, idx):
    pred_r = pred_r / jnp.linalg.norm(pred_r, axis=1, keepdims=True)
    w, x, y, z = pred_r[:, 0], pred_r[:, 1], pred_r[:, 2], pred_r[:, 3]
    rt = jnp.stack([
        1.0 - 2.0 * (y * y + z * z), 2.0 * (x * y - w * z), 2.0 * (x * z + w * y),
        2.0 * (x * y + w * z), 1.0 - 2.0 * (x * x + z * z), 2.0 * (y * z - w * x),
        2.0 * (x * z - w * y), 2.0 * (y * z + w * x), 1.0 - 2.0 * (x * x + y * y),
        pred_t[:, 0], pred_t[:, 1], pred_t[:, 2],
    ], axis=1)
    rt = jnp.pad(rt, ((0, 0), (0, 4)))

    sym = jnp.asarray(_SYM, dtype=idx.dtype)
    mask = (idx[:, 0][:, None] == sym[None, :]).any(axis=1).astype(jnp.int32)

    mp_pad = jnp.pad(model_points, ((0, 0), (0, _NPAD - _N), (0, 0)))
    t_pad = jnp.pad(target, ((0, 0), (0, _NPAD - _N), (0, 0)),
                    constant_values=_PAD_COORD)
    tT = jnp.transpose(t_pad, (0, 2, 1))

    out = _run(mask, rt, mp_pad, tT, t_pad)
    return out[:, 0, 0]


# speedup vs baseline: 3.1633x; 1.4111x over previous
import jax
import jax.numpy as jnp
from jax.experimental import pallas as pl
from jax.experimental.pallas import tpu as pltpu

_B = 64
_N = 3000
_NPAD = 3072
_JT = 768
_NJT = _NPAD // _JT
_SYM = (12, 15, 18, 19, 20)
_PAD_COORD = 1.0e6


def _loss_kernel(mask_ref, rt_ref, mp_ref, tT_ref, trows_ref, out_ref):
    b = pl.program_id(0)

    mp = mp_ref[0]
    mx, my, mz = mp[:, 0:1], mp[:, 1:2], mp[:, 2:3]

    tfx = mx * rt_ref[b, 0] + my * rt_ref[b, 3] + mz * rt_ref[b, 6] + rt_ref[b, 9]
    tfy = mx * rt_ref[b, 1] + my * rt_ref[b, 4] + mz * rt_ref[b, 7] + rt_ref[b, 10]
    tfz = mx * rt_ref[b, 2] + my * rt_ref[b, 5] + mz * rt_ref[b, 8] + rt_ref[b, 11]

    row_id = jax.lax.broadcasted_iota(jnp.int32, (_NPAD, 1), 0)
    row_ok = row_id < _N

    is_sym = mask_ref[b] != 0

    @pl.when(is_sym)
    def _sym_path():
        ones = jnp.ones_like(tfx)
        a = jnp.concatenate([-2.0 * tfx, -2.0 * tfy, -2.0 * tfz, ones], axis=1)
        q2 = tfx * tfx + tfy * tfy + tfz * tfz
        dmin = None
        for j in range(_NJT):
            tT = tT_ref[0, :, j * _JT:(j + 1) * _JT]
            r2 = (tT[0:1, :] * tT[0:1, :] + tT[1:2, :] * tT[1:2, :]
                  + tT[2:3, :] * tT[2:3, :])
            bm = jnp.concatenate([tT, r2], axis=0)
            d2m = jnp.dot(a, bm, preferred_element_type=jnp.float32,
                          precision=jax.lax.Precision.DEFAULT)
            tmin = jnp.min(d2m, axis=1, keepdims=True)
            dmin = tmin if dmin is None else jnp.minimum(dmin, tmin)
        dis = jnp.sqrt(jnp.maximum(dmin + q2, 0.0))
        out_ref[...] = jnp.sum(jnp.where(row_ok, dis, 0.0)).reshape(1, 1, 1) / _N

    @pl.when(jnp.logical_not(is_sym))
    def _direct_path():
        tr = trows_ref[0]
        dx = tfx - tr[:, 0:1]
        dy = tfy - tr[:, 1:2]
        dz = tfz - tr[:, 2:3]
        dis = jnp.sqrt(dx * dx + dy * dy + dz * dz)
        out_ref[...] = jnp.sum(jnp.where(row_ok, dis, 0.0)).reshape(1, 1, 1) / _N


@jax.jit
def _run(mask, rt, mp_pad, tT, t_pad):
    grid_spec = pltpu.PrefetchScalarGridSpec(
        num_scalar_prefetch=2,
        grid=(_B,),
        in_specs=[
            pl.BlockSpec((1, _NPAD, 3), lambda b, m, r: (b, 0, 0)),
            pl.BlockSpec((1, 3, _NPAD), lambda b, m, r: (b, 0, 0)),
            pl.BlockSpec((1, _NPAD, 3), lambda b, m, r: (b, 0, 0)),
        ],
        out_specs=pl.BlockSpec((1, 1, 1), lambda b, m, r: (b, 0, 0)),
    )
    return pl.pallas_call(
        _loss_kernel,
        grid_spec=grid_spec,
        out_shape=jax.ShapeDtypeStruct((_B, 1, 1), jnp.float32),
        compiler_params=pltpu.CompilerParams(
            dimension_semantics=("arbitrary",),
        ),
    )(mask, rt, mp_pad, tT, t_pad)


def kernel(pred_r, pred_t, target, model_points, idx):
    pred_r = pred_r / jnp.linalg.norm(pred_r, axis=1, keepdims=True)
    w, x, y, z = pred_r[:, 0], pred_r[:, 1], pred_r[:, 2], pred_r[:, 3]
    rt = jnp.stack([
        1.0 - 2.0 * (y * y + z * z), 2.0 * (x * y - w * z), 2.0 * (x * z + w * y),
        2.0 * (x * y + w * z), 1.0 - 2.0 * (x * x + z * z), 2.0 * (y * z - w * x),
        2.0 * (x * z - w * y), 2.0 * (y * z + w * x), 1.0 - 2.0 * (x * x + y * y),
        pred_t[:, 0], pred_t[:, 1], pred_t[:, 2],
    ], axis=1)
    rt = jnp.pad(rt, ((0, 0), (0, 4)))

    sym = jnp.asarray(_SYM, dtype=idx.dtype)
    mask = (idx[:, 0][:, None] == sym[None, :]).any(axis=1).astype(jnp.int32)

    mp_pad = jnp.pad(model_points, ((0, 0), (0, _NPAD - _N), (0, 0)))
    t_pad = jnp.pad(target, ((0, 0), (0, _NPAD - _N), (0, 0)),
                    constant_values=_PAD_COORD)
    tT = jnp.transpose(t_pad, (0, 2, 1))

    out = _run(mask, rt, mp_pad, tT, t_pad)
    return out[:, 0, 0]


# speedup vs baseline: 4.1544x; 1.3133x over previous
import jax
import jax.numpy as jnp
from jax.experimental import pallas as pl
from jax.experimental.pallas import tpu as pltpu

_B = 64
_N = 3000
_NPAD = 3072
_JT = 768
_NJT = _NPAD // _JT
_SYM = (12, 15, 18, 19, 20)
_PAD_COORD = 1.0e6


def _loss_kernel(mask_ref, rt_ref, mpT_ref, tgtT_ref, trows4_ref, out_ref):
    b = pl.program_id(0)

    mx = mpT_ref[0, 0:1, :]
    my = mpT_ref[0, 1:2, :]
    mz = mpT_ref[0, 2:3, :]

    tfx = mx * rt_ref[b, 0] + my * rt_ref[b, 3] + mz * rt_ref[b, 6] + rt_ref[b, 9]
    tfy = mx * rt_ref[b, 1] + my * rt_ref[b, 4] + mz * rt_ref[b, 7] + rt_ref[b, 10]
    tfz = mx * rt_ref[b, 2] + my * rt_ref[b, 5] + mz * rt_ref[b, 8] + rt_ref[b, 11]

    col_id = jax.lax.broadcasted_iota(jnp.int32, (1, _NPAD), 1)
    col_ok = col_id < _N

    is_sym = mask_ref[b] != 0

    @pl.when(is_sym)
    def _sym_path():
        yk = jnp.concatenate(
            [-2.0 * tfx, -2.0 * tfy, -2.0 * tfz, jnp.ones_like(tfx)], axis=0)
        q2 = tfx * tfx + tfy * tfy + tfz * tfz
        dmin = None
        for j in range(_NJT):
            xk = trows4_ref[0, j * _JT:(j + 1) * _JT, :]
            d2t = jnp.dot(xk, yk, preferred_element_type=jnp.float32)
            tmin = jnp.min(d2t, axis=0, keepdims=True)
            dmin = tmin if dmin is None else jnp.minimum(dmin, tmin)
        dis = jnp.sqrt(jnp.maximum(dmin + q2, 0.0))
        out_ref[...] = jnp.sum(jnp.where(col_ok, dis, 0.0)).reshape(1, 1, 1) / _N

    @pl.when(jnp.logical_not(is_sym))
    def _direct_path():
        dx = tfx - tgtT_ref[0, 0:1, :]
        dy = tfy - tgtT_ref[0, 1:2, :]
        dz = tfz - tgtT_ref[0, 2:3, :]
        dis = jnp.sqrt(dx * dx + dy * dy + dz * dz)
        out_ref[...] = jnp.sum(jnp.where(col_ok, dis, 0.0)).reshape(1, 1, 1) / _N


@jax.jit
def _run(mask, rt, mpT, tgtT, trows4):
    grid_spec = pltpu.PrefetchScalarGridSpec(
        num_scalar_prefetch=2,
        grid=(_B,),
        in_specs=[
            pl.BlockSpec((1, 3, _NPAD), lambda b, m, r: (b, 0, 0)),
            pl.BlockSpec((1, 3, _NPAD), lambda b, m, r: (b, 0, 0)),
            pl.BlockSpec((1, _NPAD, 4), lambda b, m, r: (b, 0, 0)),
        ],
        out_specs=pl.BlockSpec((1, 1, 1), lambda b, m, r: (b, 0, 0)),
    )
    return pl.pallas_call(
        _loss_kernel,
        grid_spec=grid_spec,
        out_shape=jax.ShapeDtypeStruct((_B, 1, 1), jnp.float32),
        compiler_params=pltpu.CompilerParams(
            dimension_semantics=("arbitrary",),
        ),
    )(mask, rt, mpT, tgtT, trows4)


def kernel(pred_r, pred_t, target, model_points, idx):
    pred_r = pred_r / jnp.linalg.norm(pred_r, axis=1, keepdims=True)
    w, x, y, z = pred_r[:, 0], pred_r[:, 1], pred_r[:, 2], pred_r[:, 3]
    rt = jnp.stack([
        1.0 - 2.0 * (y * y + z * z), 2.0 * (x * y - w * z), 2.0 * (x * z + w * y),
        2.0 * (x * y + w * z), 1.0 - 2.0 * (x * x + z * z), 2.0 * (y * z - w * x),
        2.0 * (x * z - w * y), 2.0 * (y * z + w * x), 1.0 - 2.0 * (x * x + y * y),
        pred_t[:, 0], pred_t[:, 1], pred_t[:, 2],
    ], axis=1)
    rt = jnp.pad(rt, ((0, 0), (0, 4)))

    sym = jnp.asarray(_SYM, dtype=idx.dtype)
    mask = (idx[:, 0][:, None] == sym[None, :]).any(axis=1).astype(jnp.int32)

    mp_pad = jnp.pad(model_points, ((0, 0), (0, _NPAD - _N), (0, 0)))
    t_pad = jnp.pad(target, ((0, 0), (0, _NPAD - _N), (0, 0)),
                    constant_values=_PAD_COORD)
    mpT = jnp.transpose(mp_pad, (0, 2, 1))
    tgtT = jnp.transpose(t_pad, (0, 2, 1))
    r2 = jnp.sum(t_pad * t_pad, axis=2, keepdims=True)
    trows4 = jnp.concatenate([t_pad, r2], axis=2)

    out = _run(mask, rt, mpT, tgtT, trows4)
    return out[:, 0, 0]


# speedup vs baseline: 18.4617x; 4.4439x over previous
import jax
import jax.numpy as jnp
from jax.experimental import pallas as pl
from jax.experimental.pallas import tpu as pltpu

_B = 64
_N = 3000
_NPAD = 3072
_JT = 768
_NJT = _NPAD // _JT
_SYM = (12, 15, 18, 19, 20)
_PAD_COORD = 1.0e6


def _loss_kernel(mask_ref, rt_ref, mpT_ref, tgtT_ref, out_ref):
    b = pl.program_id(0)

    mx = mpT_ref[0, 0:1, :]
    my = mpT_ref[0, 1:2, :]
    mz = mpT_ref[0, 2:3, :]

    tfx = mx * rt_ref[b, 0] + my * rt_ref[b, 3] + mz * rt_ref[b, 6] + rt_ref[b, 9]
    tfy = mx * rt_ref[b, 1] + my * rt_ref[b, 4] + mz * rt_ref[b, 7] + rt_ref[b, 10]
    tfz = mx * rt_ref[b, 2] + my * rt_ref[b, 5] + mz * rt_ref[b, 8] + rt_ref[b, 11]

    col_id = jax.lax.broadcasted_iota(jnp.int32, (1, _NPAD), 1)
    col_ok = col_id < _N

    is_sym = mask_ref[b] != 0

    @pl.when(is_sym)
    def _sym_path():
        yk = jnp.concatenate(
            [-2.0 * tfx, -2.0 * tfy, -2.0 * tfz, jnp.ones_like(tfx)], axis=0)
        q2 = tfx * tfx + tfy * tfy + tfz * tfz
        gx = tgtT_ref[0, 0:1, :]
        gy = tgtT_ref[0, 1:2, :]
        gz = tgtT_ref[0, 2:3, :]
        r2 = gx * gx + gy * gy + gz * gz
        dmin = None
        for j in range(_NJT):
            js = slice(j * _JT, (j + 1) * _JT)
            xkT = jnp.concatenate(
                [tgtT_ref[0, :, js], r2[:, js]], axis=0)
            d2t = jax.lax.dot_general(
                xkT, yk, (((0,), (0,)), ((), ())),
                preferred_element_type=jnp.float32)
            tmin = jnp.min(d2t, axis=0, keepdims=True)
            dmin = tmin if dmin is None else jnp.minimum(dmin, tmin)
        dis = jnp.sqrt(jnp.maximum(dmin + q2, 0.0))
        out_ref[...] = jnp.sum(jnp.where(col_ok, dis, 0.0)).reshape(1, 1, 1) / _N

    @pl.when(jnp.logical_not(is_sym))
    def _direct_path():
        dx = tfx - tgtT_ref[0, 0:1, :]
        dy = tfy - tgtT_ref[0, 1:2, :]
        dz = tfz - tgtT_ref[0, 2:3, :]
        dis = jnp.sqrt(dx * dx + dy * dy + dz * dz)
        out_ref[...] = jnp.sum(jnp.where(col_ok, dis, 0.0)).reshape(1, 1, 1) / _N


@jax.jit
def _run(mask, rt, mpT, tgtT):
    grid_spec = pltpu.PrefetchScalarGridSpec(
        num_scalar_prefetch=2,
        grid=(_B,),
        in_specs=[
            pl.BlockSpec((1, 3, _NPAD), lambda b, m, r: (b, 0, 0)),
            pl.BlockSpec((1, 3, _NPAD), lambda b, m, r: (b, 0, 0)),
        ],
        out_specs=pl.BlockSpec((1, 1, 1), lambda b, m, r: (b, 0, 0)),
    )
    return pl.pallas_call(
        _loss_kernel,
        grid_spec=grid_spec,
        out_shape=jax.ShapeDtypeStruct((_B, 1, 1), jnp.float32),
        compiler_params=pltpu.CompilerParams(
            dimension_semantics=("arbitrary",),
        ),
    )(mask, rt, mpT, tgtT)


def kernel(pred_r, pred_t, target, model_points, idx):
    pred_r = pred_r / jnp.linalg.norm(pred_r, axis=1, keepdims=True)
    w, x, y, z = pred_r[:, 0], pred_r[:, 1], pred_r[:, 2], pred_r[:, 3]
    rt = jnp.stack([
        1.0 - 2.0 * (y * y + z * z), 2.0 * (x * y - w * z), 2.0 * (x * z + w * y),
        2.0 * (x * y + w * z), 1.0 - 2.0 * (x * x + z * z), 2.0 * (y * z - w * x),
        2.0 * (x * z - w * y), 2.0 * (y * z + w * x), 1.0 - 2.0 * (x * x + y * y),
        pred_t[:, 0], pred_t[:, 1], pred_t[:, 2],
    ], axis=1)
    rt = jnp.pad(rt, ((0, 0), (0, 4)))

    sym = jnp.asarray(_SYM, dtype=idx.dtype)
    mask = (idx[:, 0][:, None] == sym[None, :]).any(axis=1).astype(jnp.int32)

    mp_pad = jnp.pad(model_points, ((0, 0), (0, _NPAD - _N), (0, 0)))
    t_pad = jnp.pad(target, ((0, 0), (0, _NPAD - _N), (0, 0)),
                    constant_values=_PAD_COORD)
    mpT = jnp.transpose(mp_pad, (0, 2, 1))
    tgtT = jnp.transpose(t_pad, (0, 2, 1))

    out = _run(mask, rt, mpT, tgtT)
    return out[:, 0, 0]


# speedup vs baseline: 18.4689x; 1.0004x over previous
import jax
import jax.numpy as jnp
from jax.experimental import pallas as pl
from jax.experimental.pallas import tpu as pltpu

_B = 64
_N = 3000
_NPAD = 3072
_JT = 768
_NJT = _NPAD // _JT
_SYM = (12, 15, 18, 19, 20)
_PAD_COORD = 1.0e6


def _loss_kernel(mask_ref, rt_ref, mpT_ref, tgtT_ref, out_ref):
    b = pl.program_id(0)

    mx = mpT_ref[0, 0:1, :]
    my = mpT_ref[0, 1:2, :]
    mz = mpT_ref[0, 2:3, :]

    tfx = mx * rt_ref[b, 0] + my * rt_ref[b, 3] + mz * rt_ref[b, 6] + rt_ref[b, 9]
    tfy = mx * rt_ref[b, 1] + my * rt_ref[b, 4] + mz * rt_ref[b, 7] + rt_ref[b, 10]
    tfz = mx * rt_ref[b, 2] + my * rt_ref[b, 5] + mz * rt_ref[b, 8] + rt_ref[b, 11]

    col_id = jax.lax.broadcasted_iota(jnp.int32, (1, _NPAD), 1)
    col_ok = col_id < _N

    is_sym = mask_ref[b] != 0

    @pl.when(is_sym)
    def _sym_path():
        ones = jnp.ones_like(tfx)
        q2 = tfx * tfx + tfy * tfy + tfz * tfz
        yk = jnp.concatenate([-2.0 * tfx, -2.0 * tfy, -2.0 * tfz, q2, ones],
                             axis=0)
        gx = tgtT_ref[0, 0:1, :]
        gy = tgtT_ref[0, 1:2, :]
        gz = tgtT_ref[0, 2:3, :]
        r2 = gx * gx + gy * gy + gz * gz
        ones_r = jnp.ones_like(r2)
        dmin = None
        for j in range(_NJT):
            js = slice(j * _JT, (j + 1) * _JT)
            xkT = jnp.concatenate(
                [tgtT_ref[0, :, js], ones_r[:, js], r2[:, js]], axis=0)
            d2t = jax.lax.dot_general(
                xkT, yk, (((0,), (0,)), ((), ())),
                preferred_element_type=jnp.float32)
            tmin = jnp.min(d2t, axis=0, keepdims=True)
            dmin = tmin if dmin is None else jnp.minimum(dmin, tmin)
        dis = jnp.sqrt(jnp.maximum(dmin, 0.0))
        out_ref[...] = jnp.sum(jnp.where(col_ok, dis, 0.0)).reshape(1, 1, 1) / _N

    @pl.when(jnp.logical_not(is_sym))
    def _direct_path():
        dx = tfx - tgtT_ref[0, 0:1, :]
        dy = tfy - tgtT_ref[0, 1:2, :]
        dz = tfz - tgtT_ref[0, 2:3, :]
        dis = jnp.sqrt(dx * dx + dy * dy + dz * dz)
        out_ref[...] = jnp.sum(jnp.where(col_ok, dis, 0.0)).reshape(1, 1, 1) / _N


@jax.jit
def _run(mask, rt, mpT, tgtT):
    grid_spec = pltpu.PrefetchScalarGridSpec(
        num_scalar_prefetch=2,
        grid=(_B,),
        in_specs=[
            pl.BlockSpec((1, 3, _NPAD), lambda b, m, r: (b, 0, 0)),
            pl.BlockSpec((1, 3, _NPAD), lambda b, m, r: (b, 0, 0)),
        ],
        out_specs=pl.BlockSpec((1, 1, 1), lambda b, m, r: (b, 0, 0)),
    )
    return pl.pallas_call(
        _loss_kernel,
        grid_spec=grid_spec,
        out_shape=jax.ShapeDtypeStruct((_B, 1, 1), jnp.float32),
        compiler_params=pltpu.CompilerParams(
            dimension_semantics=("parallel",),
        ),
    )(mask, rt, mpT, tgtT)


def kernel(pred_r, pred_t, target, model_points, idx):
    pred_r = pred_r / jnp.linalg.norm(pred_r, axis=1, keepdims=True)
    w, x, y, z = pred_r[:, 0], pred_r[:, 1], pred_r[:, 2], pred_r[:, 3]
    rt = jnp.stack([
        1.0 - 2.0 * (y * y + z * z), 2.0 * (x * y - w * z), 2.0 * (x * z + w * y),
        2.0 * (x * y + w * z), 1.0 - 2.0 * (x * x + z * z), 2.0 * (y * z - w * x),
        2.0 * (x * z - w * y), 2.0 * (y * z + w * x), 1.0 - 2.0 * (x * x + y * y),
        pred_t[:, 0], pred_t[:, 1], pred_t[:, 2],
    ], axis=1)
    rt = jnp.pad(rt, ((0, 0), (0, 4)))

    sym = jnp.asarray(_SYM, dtype=idx.dtype)
    mask = (idx[:, 0][:, None] == sym[None, :]).any(axis=1).astype(jnp.int32)

    mp_pad = jnp.pad(model_points, ((0, 0), (0, _NPAD - _N), (0, 0)))
    t_pad = jnp.pad(target, ((0, 0), (0, _NPAD - _N), (0, 0)),
                    constant_values=_PAD_COORD)
    mpT = jnp.transpose(mp_pad, (0, 2, 1))
    tgtT = jnp.transpose(t_pad, (0, 2, 1))

    out = _run(mask, rt, mpT, tgtT)
    return out[:, 0, 0]
